# Initial kernel scaffold; baseline (speedup 1.0000x reference)
import jax
import jax.numpy as jnp
from jax.experimental import pallas as pl
from jax.experimental.pallas import tpu as pltpu

EK = 2048
NB = 25
NRP = 208


def _rmpi_tc_kernel(u_ref, v_ref, lbl_ref, dst_ref, src_ref, typ_ref,
                    rel_ref, w_ref, b_ref, fcw_ref, fcb_ref, out_ref, g_ref):
    i = pl.program_id(0)

    @pl.when(i == 0)
    def _init():
        f = fcw_ref[0:1, :]
        cols = []
        for j in range(6):
            wj = w_ref[32 * j:32 * j + 32, :]
            vj = jax.lax.dot_general(f, wj, (((1,), (0,)), ((), ())))
            tj = jax.lax.dot_general(rel_ref[...], vj, (((1,), (1,)), ((), ())))
            bj = jnp.sum(b_ref[j:j + 1, :] * f)
            cols.append(tj + bj)
        r = jax.lax.dot_general(rel_ref[...], f, (((1,), (1,)), ((), ())))
        g4p = cols[4] - cols[1] - cols[2]
        g5p = cols[5] - cols[0] - cols[3]
        g_ref[...] = jnp.concatenate(
            [cols[0], cols[1], cols[2], cols[3], g4p, g5p, r, jnp.zeros_like(r)],
            axis=1)
        oh = (lbl_ref[...] == jax.lax.broadcasted_iota(jnp.int32, (1, NRP), 1)
              ).astype(jnp.float32)
        out_ref[...] = jax.lax.dot_general(
            oh, g_ref[:, 6:7], (((1,), (0,)), ((), ()))) + fcb_ref[0, 0]

    u = u_ref[...]
    v = v_ref[...]
    dst = dst_ref[0]
    src = src_ref[0]
    tcol = typ_ref[0]
    a = (dst == u).astype(jnp.float32)
    bq = (src == u).astype(jnp.float32)
    c = (dst == v).astype(jnp.float32)
    d = (src == v).astype(jnp.float32)
    m5 = bq * c
    m6 = a * d
    oh = (tcol == jax.lax.broadcasted_iota(jnp.int32, (1, NRP), 1)
          ).astype(jnp.float32)
    ge = jax.lax.dot_general(oh, g_ref[...], (((1,), (0,)), ((), ())))

    def mv(m, j):
        return jax.lax.dot_general(m, ge[:, j:j + 1], (((1,), (0,)), ((), ())))

    acc = mv(a, 0) + mv(bq, 1) + mv(c, 2) + mv(d, 3) + mv(m5, 4) + mv(m6, 5)
    out_ref[...] += acc


def kernel(edge_index, edge_type, target_u, target_v, rel_labels,
           rel_emb_weight, W_reld2, b_reld2, fc_W, fc_b):
    ep = NB * EK
    e0 = edge_type.shape[0]
    pad = ep - e0
    src = jnp.pad(edge_index[0].astype(jnp.int32), (0, pad), constant_values=-1)
    dst = jnp.pad(edge_index[1].astype(jnp.int32), (0, pad), constant_values=-1)
    typ = jnp.pad(edge_type.astype(jnp.int32), (0, pad))
    srcp = src.reshape(NB, 1, EK)
    dstp = dst.reshape(NB, 1, EK)
    typc = typ.reshape(NB, EK, 1)
    u2 = target_u.astype(jnp.int32).reshape(-1, 1)
    v2 = target_v.astype(jnp.int32).reshape(-1, 1)
    l2 = rel_labels.astype(jnp.int32).reshape(-1, 1)
    relp = jnp.pad(rel_emb_weight, ((0, NRP - rel_emb_weight.shape[0]), (0, 0)))
    wf = W_reld2.reshape(192, 32)
    bp = jnp.pad(b_reld2, ((0, 2), (0, 0)))
    fcwp = jnp.pad(fc_W, ((0, 7), (0, 0)))
    fcbp = jnp.pad(fc_b.reshape(1, 1), ((0, 7), (0, 31)))

    out = pl.pallas_call(
        _rmpi_tc_kernel,
        grid=(NB,),
        in_specs=[
            pl.BlockSpec((128, 1), lambda i: (0, 0)),
            pl.BlockSpec((128, 1), lambda i: (0, 0)),
            pl.BlockSpec((128, 1), lambda i: (0, 0)),
            pl.BlockSpec((1, 1, EK), lambda i: (i, 0, 0)),
            pl.BlockSpec((1, 1, EK), lambda i: (i, 0, 0)),
            pl.BlockSpec((1, EK, 1), lambda i: (i, 0, 0)),
            pl.BlockSpec((NRP, 32), lambda i: (0, 0)),
            pl.BlockSpec((192, 32), lambda i: (0, 0)),
            pl.BlockSpec((8, 32), lambda i: (0, 0)),
            pl.BlockSpec((8, 32), lambda i: (0, 0)),
            pl.BlockSpec((8, 32), lambda i: (0, 0)),
        ],
        out_specs=pl.BlockSpec((128, 1), lambda i: (0, 0)),
        out_shape=jax.ShapeDtypeStruct((target_u.shape[0], 1), jnp.float32),
        scratch_shapes=[pltpu.VMEM((NRP, 8), jnp.float32)],
    )(u2, v2, l2, dstp, srcp, typc, relp, wf, bp, fcwp, fcbp)
    return out


# speedup vs baseline: 6.2052x; 6.2052x over previous
import numpy as np
import jax
import jax.numpy as jnp
from jax.experimental import pallas as pl
from jax.experimental.pallas import tpu as pltpu

EK = 2048
NB = 25
NRP = 208


def _rmpi_tc_kernel(u_ref, v_ref, lbl_ref, dst_ref, src_ref, typ_ref,
                    relaug_ref, waug_ref, fcw_ref, fcb_ref, fixt_ref,
                    out_ref, vt_ref, gt_ref):
    i = pl.program_id(0)

    @pl.when(i == 0)
    def _init():
        f = fcw_ref[0:1, :]
        for j in range(6):
            wj = waug_ref[32 * j:32 * j + 32, :]
            vt_ref[j:j + 1, :] = jax.lax.dot_general(
                f, wj, (((1,), (0,)), ((), ())))
        f40 = jnp.concatenate([f, jnp.zeros((1, 8), jnp.float32)], axis=1)
        vt_ref[6:7, :] = f40
        vt_ref[7:8, :] = jnp.zeros((1, 40), jnp.float32)
        gt8 = jax.lax.dot_general(
            vt_ref[...], relaug_ref[...], (((1,), (1,)), ((), ())))
        gt_ref[...] = jax.lax.dot_general(
            fixt_ref[...], gt8, (((1,), (0,)), ((), ())))
        oh = (lbl_ref[...] == jax.lax.broadcasted_iota(jnp.int32, (1, NRP), 1)
              ).astype(jnp.float32)
        base8 = jax.lax.dot_general(
            oh, gt_ref[...], (((1,), (1,)), ((), ())))
        out_ref[...] = base8[:, 6:7] + fcb_ref[0, 0]

    u = u_ref[...]
    v = v_ref[...]
    dst = dst_ref[0]
    src = src_ref[0]
    trow = typ_ref[0]
    oht = (jax.lax.broadcasted_iota(jnp.int32, (NRP, 1), 0) == trow
           ).astype(jnp.float32)
    ger = jax.lax.dot_general(
        gt_ref[...], oht, (((1,), (0,)), ((), ())))

    a = (dst == u).astype(jnp.float32)
    bq = (src == u).astype(jnp.float32)
    c = (dst == v).astype(jnp.float32)
    d = (src == v).astype(jnp.float32)
    combined = (a * ger[0:1, :] + bq * ger[1:2, :]
                + c * ger[2:3, :] + d * ger[3:4, :]
                + (bq * c) * ger[4:5, :] + (a * d) * ger[5:6, :])
    out_ref[...] += jnp.sum(combined, axis=1, keepdims=True)


def kernel(edge_index, edge_type, target_u, target_v, rel_labels,
           rel_emb_weight, W_reld2, b_reld2, fc_W, fc_b):
    ep = NB * EK
    e0 = edge_type.shape[0]
    pad = ep - e0
    src = jnp.pad(edge_index[0].astype(jnp.int32), (0, pad), constant_values=-1)
    dst = jnp.pad(edge_index[1].astype(jnp.int32), (0, pad), constant_values=-1)
    typ = jnp.pad(edge_type.astype(jnp.int32), (0, pad))
    srcp = src.reshape(NB, 1, EK)
    dstp = dst.reshape(NB, 1, EK)
    typp = typ.reshape(NB, 1, EK)
    u2 = target_u.astype(jnp.int32).reshape(-1, 1)
    v2 = target_v.astype(jnp.int32).reshape(-1, 1)
    l2 = rel_labels.astype(jnp.int32).reshape(-1, 1)
    nr = rel_emb_weight.shape[0]
    relaug = jnp.pad(
        jnp.concatenate([rel_emb_weight,
                         jnp.ones((nr, 1), jnp.float32)], axis=1),
        ((0, NRP - nr), (0, 7)))
    waug = jnp.concatenate(
        [W_reld2, b_reld2[:, :, None],
         jnp.zeros((6, 32, 7), jnp.float32)], axis=2).reshape(192, 40)
    fcwp = jnp.pad(fc_W, ((0, 7), (0, 0)))
    fcbp = jnp.pad(fc_b.reshape(1, 1), ((0, 7), (0, 31)))
    fix = np.eye(8, dtype=np.float32)
    fix[1, 4] = fix[2, 4] = -1.0
    fix[0, 5] = fix[3, 5] = -1.0
    fixt = jnp.asarray(fix.T)

    out = pl.pallas_call(
        _rmpi_tc_kernel,
        grid=(NB,),
        in_specs=[
            pl.BlockSpec((128, 1), lambda i: (0, 0)),
            pl.BlockSpec((128, 1), lambda i: (0, 0)),
            pl.BlockSpec((128, 1), lambda i: (0, 0)),
            pl.BlockSpec((1, 1, EK), lambda i: (i, 0, 0)),
            pl.BlockSpec((1, 1, EK), lambda i: (i, 0, 0)),
            pl.BlockSpec((1, 1, EK), lambda i: (i, 0, 0)),
            pl.BlockSpec((NRP, 40), lambda i: (0, 0)),
            pl.BlockSpec((192, 40), lambda i: (0, 0)),
            pl.BlockSpec((8, 32), lambda i: (0, 0)),
            pl.BlockSpec((8, 32), lambda i: (0, 0)),
            pl.BlockSpec((8, 8), lambda i: (0, 0)),
        ],
        out_specs=pl.BlockSpec((128, 1), lambda i: (0, 0)),
        out_shape=jax.ShapeDtypeStruct((target_u.shape[0], 1), jnp.float32),
        scratch_shapes=[pltpu.VMEM((8, 40), jnp.float32),
                        pltpu.VMEM((8, NRP), jnp.float32)],
    )(u2, v2, l2, dstp, srcp, typp, relaug, waug, fcwp, fcbp, fixt)
    return out
